# Initial kernel scaffold; baseline (speedup 1.0000x reference)
import jax
import jax.numpy as jnp
from jax.experimental import pallas as pl


def kernel(mixed_vcf, ref_panel, weights):
    raise NotImplementedError("write your pallas kernel here")


# speedup vs baseline: 17.6019x; 17.6019x over previous
import functools

import jax
import jax.numpy as jnp
from jax import lax
from jax.experimental import pallas as pl
from jax.experimental.pallas import tpu as pltpu
from jax.experimental.pallas import tpu_sc as plsc

NC = 2
NS = 16
LANES = 16
NW = NC * NS


def _make_sc_kernel(P, R, L):
    CW = 512
    LSPAN = L // NW
    CPP = LSPAN // CW
    TOTAL = P * CPP

    mesh = plsc.VectorSubcoreMesh(
        core_axis_name="c", subcore_axis_name="s",
        num_cores=NC, num_subcores=NS)

    @functools.partial(
        pl.kernel,
        out_type=[
            jax.ShapeDtypeStruct((P, L), jnp.float32),
            jax.ShapeDtypeStruct((P, L), jnp.int32),
        ],
        mesh=mesh,
        scratch_types=[
            pltpu.VMEM((2, R, CW), jnp.float32),
            pltpu.VMEM((2, CW), jnp.float32),
            pltpu.VMEM((2, CW), jnp.float32),
            pltpu.VMEM((2, CW), jnp.int32),
            pltpu.VMEM((2 * LANES,), jnp.float32),
            pltpu.SemaphoreType.DMA,
            pltpu.SemaphoreType.DMA,
            pltpu.SemaphoreType.DMA,
            pltpu.SemaphoreType.DMA,
        ],
    )
    def sc_kernel(mix_hbm, ref_hbm, w_hbm, pool_hbm, idx_hbm,
                  ref_buf, mix_buf, pool_buf, idx_buf, w_buf,
                  isem0, isem1, osem0, osem1):
        isems = (isem0, isem1)
        osems = (osem0, osem1)
        wid = lax.axis_index("s") * NC + lax.axis_index("c")
        base = wid * LSPAN

        pltpu.sync_copy(w_hbm, w_buf)
        w0 = w_buf[pl.ds(0, LANES)]
        w1 = w_buf[pl.ds(LANES, LANES)]

        def coords(g):
            return g // CPP, base + (g % CPP) * CW

        def issue_in(g, b):
            p, l0 = coords(g)
            pltpu.async_copy(ref_hbm.at[p, :, pl.ds(l0, CW)], ref_buf.at[b],
                             isems[b])
            pltpu.async_copy(mix_hbm.at[p, pl.ds(l0, CW)], mix_buf.at[b],
                             isems[b])

        def wait_in(b):
            pltpu.make_async_copy(ref_hbm.at[0, :, pl.ds(0, CW)],
                                  ref_buf.at[b], isems[b]).wait()
            pltpu.make_async_copy(mix_hbm.at[0, pl.ds(0, CW)],
                                  mix_buf.at[b], isems[b]).wait()

        def issue_out(g, b):
            p, l0 = coords(g)
            pltpu.async_copy(pool_buf.at[b], pool_hbm.at[p, pl.ds(l0, CW)],
                             osems[b])
            pltpu.async_copy(idx_buf.at[b], idx_hbm.at[p, pl.ds(l0, CW)],
                             osems[b])

        def wait_out(b):
            pltpu.make_async_copy(pool_buf.at[b],
                                  pool_hbm.at[0, pl.ds(0, CW)],
                                  osems[b]).wait()
            pltpu.make_async_copy(idx_buf.at[b],
                                  idx_hbm.at[0, pl.ds(0, CW)],
                                  osems[b]).wait()

        def compute(b):
            def jbody(j, carry):
                off = j * LANES
                mix = mix_buf[b, pl.ds(off, LANES)]
                m1 = mix * ref_buf[b, 0, pl.ds(off, LANES)]
                m2 = jnp.full((LANES,), -jnp.inf, jnp.float32)
                idx = jnp.zeros((LANES,), jnp.int32)
                for r in range(1, R):
                    v = mix * ref_buf[b, r, pl.ds(off, LANES)]
                    gt = v > m1
                    m2 = jnp.maximum(m2, jnp.where(gt, m1, v))
                    idx = jnp.where(gt, jnp.full((LANES,), r, jnp.int32), idx)
                    m1 = jnp.where(gt, v, m1)
                pool_buf[b, pl.ds(off, LANES)] = w0 * m1 + w1 * m2
                idx_buf[b, pl.ds(off, LANES)] = idx
                return carry
            lax.fori_loop(0, CW // LANES, jbody, 0)

        issue_in(0, 0)

        def outer(g2, carry):
            for bb in range(2):
                g = g2 * 2 + bb

                @pl.when(g + 1 < TOTAL)
                def _():
                    issue_in(g + 1, 1 - bb)

                wait_in(bb)

                @pl.when(g >= 2)
                def _():
                    wait_out(bb)

                compute(bb)
                issue_out(g, bb)
            return carry

        lax.fori_loop(0, TOTAL // 2, outer, 0)
        wait_out(0)
        wait_out(1)

    return sc_kernel


def kernel(mixed_vcf, ref_panel, weights):
    B, A, R, L = ref_panel.shape
    P = B * A
    ref3 = ref_panel.reshape(P, R, L)
    mix2 = jnp.broadcast_to(mixed_vcf[:, None, :], (B, A, L)).reshape(P, L)
    w_flat = jnp.repeat(weights.reshape(-1).astype(jnp.float32), LANES)
    pool, idx = _make_sc_kernel(P, R, L)(mix2, ref3, w_flat)
    return pool.reshape(B, A, L), idx.reshape(B, A, L)
